# Initial kernel scaffold; baseline (speedup 1.0000x reference)
import jax
import jax.numpy as jnp
from jax.experimental import pallas as pl


def kernel(xyz, points, time_stamps, te_w1, te_b1, te_w2, te_b2, tf_w1, tf_b1, tf_w2, tf_b2, c1_w, c1_b, c2_w, c2_b, c3_w, c3_b, bn1_g, bn1_b, bn2_g, bn2_b, bn3_g, bn3_b, q_w, q_b, k_w, k_b, v_w, v_b, o_w, o_b, sw, ln_g, ln_b):
    raise NotImplementedError("write your pallas kernel here")


# speedup vs baseline: 1.0015x; 1.0015x over previous
import math

import jax
import jax.numpy as jnp
from jax.experimental import pallas as pl

B, N, C_IN = 4, 16384, 128
NPOINT, NSAMPLE = 1024, 32
MLP_CH = (128, 128, 256)
TFD = 8
HEADS = 4


def _lin(x, w, b):
    return x @ w.T + b


def _conv1x1(x, w, b):
    return jnp.einsum('oc,bchw->bohw', w, x) + b[None, :, None, None]


def _bn(x, g, be):
    m = x.mean(axis=(0, 2, 3), keepdims=True)
    v = x.var(axis=(0, 2, 3), keepdims=True)
    return (x - m) / jnp.sqrt(v + 1e-5) * g[None, :, None, None] + be[None, :, None, None]


def _interp_linear(x, size):
    L = x.shape[-1]
    src = (jnp.arange(size, dtype=jnp.float32) + 0.5) * (L / size) - 0.5
    src = jnp.clip(src, 0.0, L - 1.0)
    i0 = jnp.floor(src).astype(jnp.int32)
    i1 = jnp.minimum(i0 + 1, L - 1)
    w = src - i0.astype(jnp.float32)
    return x[..., i0] * (1.0 - w) + x[..., i1] * w


def _ln_kernel(y_ref, g_ref, b_ref, o_ref):
    y = y_ref[...]
    mu = jnp.mean(y, axis=-1, keepdims=True)
    var = jnp.mean((y - mu) ** 2, axis=-1, keepdims=True)
    o_ref[...] = (y - mu) / jnp.sqrt(var + 1e-5) * g_ref[...] + b_ref[...]


def _layernorm_pallas(y, g, b):
    Bq, P, C = y.shape
    grid = (Bq, P // 256)
    return pl.pallas_call(
        _ln_kernel,
        grid=grid,
        in_specs=[
            pl.BlockSpec((1, 256, C), lambda i, j: (i, j, 0)),
            pl.BlockSpec((C,), lambda i, j: (0,)),
            pl.BlockSpec((C,), lambda i, j: (0,)),
        ],
        out_specs=pl.BlockSpec((1, 256, C), lambda i, j: (i, j, 0)),
        out_shape=jax.ShapeDtypeStruct((Bq, P, C), jnp.float32),
    )(y, g, b)


def kernel(xyz, points, time_stamps, te_w1, te_b1, te_w2, te_b2, tf_w1, tf_b1, tf_w2, tf_b2, c1_w, c1_b, c2_w, c2_b, c3_w, c3_b, bn1_g, bn1_b, bn2_g, bn2_b, bn3_g, bn3_b, q_w, q_b, k_w, k_b, v_w, v_b, o_w, o_b, sw, ln_g, ln_b):
    Bq = xyz.shape[0]
    t = time_stamps[:, 0, :]
    tw = jax.nn.softmax(t * 10.0, axis=-1) + 0.1
    u = jax.random.uniform(jax.random.key(42), t.shape, jnp.float32, 1e-10, 1.0)
    gumbel = -jnp.log(-jnp.log(u))
    idx = jax.lax.top_k(jnp.log(tw) + gumbel, NPOINT)[1]
    new_xyz = jnp.take_along_axis(xyz, idx[:, None, :], axis=2)
    a = new_xyz.transpose(0, 2, 1)
    c = xyz.transpose(0, 2, 1)
    d2 = jnp.sum(a * a, -1)[:, :, None] + jnp.sum(c * c, -1)[:, None, :] - 2.0 * jnp.einsum('bid,bjd->bij', a, c)
    dist = jnp.sqrt(jnp.maximum(d2, 1e-12))
    nidx = jax.lax.top_k(-dist, NSAMPLE)[1]
    grouped_xyz = jnp.take_along_axis(xyz[:, :, None, :], nidx[:, None, :, :], axis=3)
    grouped_xyz = (grouped_xyz - new_xyz[:, :, :, None]).transpose(0, 1, 3, 2)
    tf = jax.nn.relu(_lin(t[..., None], te_w1, te_b1))
    tf = _lin(tf, te_w2, te_b2)
    div = jnp.exp(jnp.arange(0, TFD, 2, dtype=jnp.float32) * (-(math.log(10000.0) / TFD)))
    pos = t[..., None] * div
    pe = jnp.stack([jnp.sin(pos), jnp.cos(pos)], axis=-1).reshape(Bq, t.shape[1], TFD)
    tf = tf + pe
    comb = jnp.concatenate([points.transpose(0, 2, 1), tf], axis=-1)
    enh = _lin(jax.nn.relu(_lin(comb, tf_w1, tf_b1)), tf_w2, tf_b2).transpose(0, 2, 1)
    gp = jnp.take_along_axis(enh[:, :, None, :], nidx[:, None, :, :], axis=3)
    np_in = jnp.concatenate([grouped_xyz, gp.transpose(0, 1, 3, 2)], axis=1)
    h = jax.nn.relu(_bn(_conv1x1(np_in, c1_w, c1_b), bn1_g, bn1_b))
    h = jax.nn.relu(_bn(_conv1x1(h, c2_w, c2_b), bn2_g, bn2_b))
    h = _bn(_conv1x1(h, c3_w, c3_b), bn3_g, bn3_b)
    pooled = jnp.max(h, axis=2)
    x_in = pooled.transpose(0, 2, 1)
    hd = MLP_CH[2] // HEADS
    outs = []
    for i, scale in enumerate((4, 8, 16)):
        nw = NPOINT // scale
        wx = x_in[:, :nw * scale, :].reshape(Bq, nw, scale, MLP_CH[2]).mean(axis=2)
        q = _lin(wx, q_w, q_b).reshape(Bq, nw, HEADS, hd).transpose(0, 2, 1, 3)
        kk = _lin(wx, k_w, k_b).reshape(Bq, nw, HEADS, hd).transpose(0, 2, 1, 3)
        vv = _lin(wx, v_w, v_b).reshape(Bq, nw, HEADS, hd).transpose(0, 2, 1, 3)
        sc = jnp.einsum('bhid,bhjd->bhij', q, kk) / math.sqrt(hd)
        aw = jax.nn.softmax(sc, axis=-1)
        att = jnp.einsum('bhij,bhjd->bhid', aw, vv).transpose(0, 2, 1, 3).reshape(Bq, nw, MLP_CH[2])
        att = _lin(att, o_w, o_b)
        if nw < NPOINT:
            att = _interp_linear(att.transpose(0, 2, 1), NPOINT).transpose(0, 2, 1)
        outs.append(att * sw[i])
    ms = (outs[0] + outs[1] + outs[2]) / 3.0
    y = x_in + ms
    y = _layernorm_pallas(y, ln_g, ln_b)
    return new_xyz, y.transpose(0, 2, 1)


# speedup vs baseline: 1.7945x; 1.7918x over previous
import math

import jax
import jax.numpy as jnp
from jax.experimental import pallas as pl

B, N, C_IN = 4, 16384, 128
NPOINT, NSAMPLE = 1024, 32
MLP_CH = (128, 128, 256)
TFD = 8
HEADS = 4


def _lin(x, w, b):
    return x @ w.T + b


def _conv1x1(x, w, b):
    return jnp.einsum('oc,bchw->bohw', w, x) + b[None, :, None, None]


def _bn(x, g, be):
    m = x.mean(axis=(0, 2, 3), keepdims=True)
    v = x.var(axis=(0, 2, 3), keepdims=True)
    return (x - m) / jnp.sqrt(v + 1e-5) * g[None, :, None, None] + be[None, :, None, None]


def _interp_linear(x, size):
    L = x.shape[-1]
    src = (jnp.arange(size, dtype=jnp.float32) + 0.5) * (L / size) - 0.5
    src = jnp.clip(src, 0.0, L - 1.0)
    i0 = jnp.floor(src).astype(jnp.int32)
    i1 = jnp.minimum(i0 + 1, L - 1)
    w = src - i0.astype(jnp.float32)
    return x[..., i0] * (1.0 - w) + x[..., i1] * w


KNN_TILE_P = 128
KNN_CH = 128


def _knn_kernel(xyz_ref, nxyz_ref, out_ref):
    TILE_P, CH, S = KNN_TILE_P, KNN_CH, NSAMPLE
    NCH = N // CH
    xyz = xyz_ref[0]
    a = nxyz_ref[0]
    cj2 = jnp.sum(xyz * xyz, axis=0, keepdims=True)
    dots = jax.lax.dot_general(a, xyz, (((0,), (0,)), ((), ())),
                               preferred_element_type=jnp.float32)
    d = cj2 - 2.0 * dots
    D3 = d.reshape(TILE_P, CH, NCH)
    cmin0 = jnp.min(D3, axis=1)
    chunkiota = jax.lax.broadcasted_iota(jnp.int32, (TILE_P, NCH), 1)
    laneiota = jax.lax.broadcasted_iota(jnp.int32, (TILE_P, CH), 1)
    slotiota = jax.lax.broadcasted_iota(jnp.int32, (TILE_P, S), 1)

    def body(i, carry):
        cmin, sel = carry
        cstar = jnp.argmin(cmin, axis=-1).astype(jnp.int32)
        idx3 = jnp.broadcast_to(cstar[:, None, None], (TILE_P, CH, 1))
        contents = jnp.take_along_axis(D3, idx3, axis=2)[:, :, 0]
        gid = laneiota * NCH + cstar[:, None]
        hit = jnp.any(gid[:, :, None] == sel[:, None, :], axis=-1)
        contents = jnp.where(hit, jnp.inf, contents)
        lstar = jnp.argmin(contents, axis=-1).astype(jnp.int32)
        j = lstar * NCH + cstar
        sel = jnp.where(slotiota == i, j[:, None], sel)
        c2 = jnp.where(laneiota == lstar[:, None], jnp.inf, contents)
        newmin = jnp.min(c2, axis=-1)
        cmin = jnp.where(chunkiota == cstar[:, None], newmin[:, None], cmin)
        return cmin, sel

    sel0 = jnp.full((TILE_P, S), -1, jnp.int32)
    _, sel = jax.lax.fori_loop(0, S, body, (cmin0, sel0))
    out_ref[0] = sel


def _knn_pallas(xyz, new_xyz):
    return pl.pallas_call(
        _knn_kernel,
        grid=(B, NPOINT // KNN_TILE_P),
        in_specs=[
            pl.BlockSpec((1, 3, N), lambda b, p: (b, 0, 0)),
            pl.BlockSpec((1, 3, KNN_TILE_P), lambda b, p: (b, 0, p)),
        ],
        out_specs=pl.BlockSpec((1, KNN_TILE_P, NSAMPLE), lambda b, p: (b, p, 0)),
        out_shape=jax.ShapeDtypeStruct((B, NPOINT, NSAMPLE), jnp.int32),
    )(xyz, new_xyz)


def _ln_kernel(y_ref, g_ref, b_ref, o_ref):
    y = y_ref[...]
    mu = jnp.mean(y, axis=-1, keepdims=True)
    var = jnp.mean((y - mu) ** 2, axis=-1, keepdims=True)
    o_ref[...] = (y - mu) / jnp.sqrt(var + 1e-5) * g_ref[...] + b_ref[...]


def _layernorm_pallas(y, g, b):
    Bq, P, C = y.shape
    grid = (Bq, P // 256)
    return pl.pallas_call(
        _ln_kernel,
        grid=grid,
        in_specs=[
            pl.BlockSpec((1, 256, C), lambda i, j: (i, j, 0)),
            pl.BlockSpec((C,), lambda i, j: (0,)),
            pl.BlockSpec((C,), lambda i, j: (0,)),
        ],
        out_specs=pl.BlockSpec((1, 256, C), lambda i, j: (i, j, 0)),
        out_shape=jax.ShapeDtypeStruct((Bq, P, C), jnp.float32),
    )(y, g, b)


def kernel(xyz, points, time_stamps, te_w1, te_b1, te_w2, te_b2, tf_w1, tf_b1, tf_w2, tf_b2, c1_w, c1_b, c2_w, c2_b, c3_w, c3_b, bn1_g, bn1_b, bn2_g, bn2_b, bn3_g, bn3_b, q_w, q_b, k_w, k_b, v_w, v_b, o_w, o_b, sw, ln_g, ln_b):
    Bq = xyz.shape[0]
    t = time_stamps[:, 0, :]
    tw = jax.nn.softmax(t * 10.0, axis=-1) + 0.1
    u = jax.random.uniform(jax.random.key(42), t.shape, jnp.float32, 1e-10, 1.0)
    gumbel = -jnp.log(-jnp.log(u))
    idx = jax.lax.top_k(jnp.log(tw) + gumbel, NPOINT)[1]
    new_xyz = jnp.take_along_axis(xyz, idx[:, None, :], axis=2)
    nidx = _knn_pallas(xyz, new_xyz)
    grouped_xyz = jnp.take_along_axis(xyz[:, :, None, :], nidx[:, None, :, :], axis=3)
    grouped_xyz = (grouped_xyz - new_xyz[:, :, :, None]).transpose(0, 1, 3, 2)
    tf = jax.nn.relu(_lin(t[..., None], te_w1, te_b1))
    tf = _lin(tf, te_w2, te_b2)
    div = jnp.exp(jnp.arange(0, TFD, 2, dtype=jnp.float32) * (-(math.log(10000.0) / TFD)))
    pos = t[..., None] * div
    pe = jnp.stack([jnp.sin(pos), jnp.cos(pos)], axis=-1).reshape(Bq, t.shape[1], TFD)
    tf = tf + pe
    comb = jnp.concatenate([points.transpose(0, 2, 1), tf], axis=-1)
    enh = _lin(jax.nn.relu(_lin(comb, tf_w1, tf_b1)), tf_w2, tf_b2).transpose(0, 2, 1)
    gp = jnp.take_along_axis(enh[:, :, None, :], nidx[:, None, :, :], axis=3)
    np_in = jnp.concatenate([grouped_xyz, gp.transpose(0, 1, 3, 2)], axis=1)
    h = jax.nn.relu(_bn(_conv1x1(np_in, c1_w, c1_b), bn1_g, bn1_b))
    h = jax.nn.relu(_bn(_conv1x1(h, c2_w, c2_b), bn2_g, bn2_b))
    h = _bn(_conv1x1(h, c3_w, c3_b), bn3_g, bn3_b)
    pooled = jnp.max(h, axis=2)
    x_in = pooled.transpose(0, 2, 1)
    hd = MLP_CH[2] // HEADS
    outs = []
    for i, scale in enumerate((4, 8, 16)):
        nw = NPOINT // scale
        wx = x_in[:, :nw * scale, :].reshape(Bq, nw, scale, MLP_CH[2]).mean(axis=2)
        q = _lin(wx, q_w, q_b).reshape(Bq, nw, HEADS, hd).transpose(0, 2, 1, 3)
        kk = _lin(wx, k_w, k_b).reshape(Bq, nw, HEADS, hd).transpose(0, 2, 1, 3)
        vv = _lin(wx, v_w, v_b).reshape(Bq, nw, HEADS, hd).transpose(0, 2, 1, 3)
        sc = jnp.einsum('bhid,bhjd->bhij', q, kk) / math.sqrt(hd)
        aw = jax.nn.softmax(sc, axis=-1)
        att = jnp.einsum('bhij,bhjd->bhid', aw, vv).transpose(0, 2, 1, 3).reshape(Bq, nw, MLP_CH[2])
        att = _lin(att, o_w, o_b)
        if nw < NPOINT:
            att = _interp_linear(att.transpose(0, 2, 1), NPOINT).transpose(0, 2, 1)
        outs.append(att * sw[i])
    ms = (outs[0] + outs[1] + outs[2]) / 3.0
    y = x_in + ms
    y = _layernorm_pallas(y, ln_g, ln_b)
    return new_xyz, y.transpose(0, 2, 1)


# speedup vs baseline: 3.8349x; 2.1370x over previous
import math

import jax
import jax.numpy as jnp
from jax.experimental import pallas as pl

B, N, C_IN = 4, 16384, 128
NPOINT, NSAMPLE = 1024, 32
MLP_CH = (128, 128, 256)
TFD = 8
HEADS = 4


def _lin(x, w, b):
    return x @ w.T + b


def _conv1x1(x, w, b):
    return jnp.einsum('oc,bchw->bohw', w, x) + b[None, :, None, None]


def _bn(x, g, be):
    m = x.mean(axis=(0, 2, 3), keepdims=True)
    v = x.var(axis=(0, 2, 3), keepdims=True)
    return (x - m) / jnp.sqrt(v + 1e-5) * g[None, :, None, None] + be[None, :, None, None]


def _interp_linear(x, size):
    L = x.shape[-1]
    src = (jnp.arange(size, dtype=jnp.float32) + 0.5) * (L / size) - 0.5
    src = jnp.clip(src, 0.0, L - 1.0)
    i0 = jnp.floor(src).astype(jnp.int32)
    i1 = jnp.minimum(i0 + 1, L - 1)
    w = src - i0.astype(jnp.float32)
    return x[..., i0] * (1.0 - w) + x[..., i1] * w


KNN_TILE_P = 128
KNN_CH = 128


def _knn_kernel(xyz_ref, nxyz_ref, out_ref):
    TILE_P, S = KNN_TILE_P, NSAMPLE
    NCH = 8
    CH = N // NCH
    xyz = xyz_ref[0]
    a = nxyz_ref[0]
    cj2 = jnp.sum(xyz * xyz, axis=0, keepdims=True)
    dots = jax.lax.dot_general(a, xyz, (((0,), (0,)), ((), ())),
                               preferred_element_type=jnp.float32)
    d = cj2 - 2.0 * dots
    D3 = d.reshape(TILE_P, NCH, CH)
    cmin0 = jnp.min(D3, axis=2)
    chunkiota = jax.lax.broadcasted_iota(jnp.int32, (TILE_P, NCH), 1)
    laneiota = jax.lax.broadcasted_iota(jnp.int32, (TILE_P, CH), 1)
    slotiota = jax.lax.broadcasted_iota(jnp.int32, (TILE_P, S), 1)

    def body(i, carry):
        cmin, selbits, out = carry
        cstar = jnp.argmin(cmin, axis=-1).astype(jnp.int32)
        idx3 = jnp.broadcast_to(cstar[:, None, None], (TILE_P, 1, CH))
        contents = jnp.take_along_axis(D3, idx3, axis=1)[:, 0, :]
        hit = jax.lax.shift_right_logical(selbits, cstar[:, None]) & 1
        contents = jnp.where(hit == 1, jnp.inf, contents)
        lstar = jnp.argmin(contents, axis=-1).astype(jnp.int32)
        j = cstar * CH + lstar
        out = jnp.where(slotiota == i, j[:, None], out)
        selbits = selbits | jnp.where(
            laneiota == lstar[:, None],
            jax.lax.shift_left(jnp.ones_like(selbits), cstar[:, None]),
            jnp.zeros_like(selbits))
        c2 = jnp.where(laneiota == lstar[:, None], jnp.inf, contents)
        newmin = jnp.min(c2, axis=-1)
        cmin = jnp.where(chunkiota == cstar[:, None], newmin[:, None], cmin)
        return cmin, selbits, out

    selbits0 = jnp.zeros((TILE_P, CH), jnp.int32)
    out0 = jnp.zeros((TILE_P, S), jnp.int32)
    _, _, sel = jax.lax.fori_loop(0, S, body, (cmin0, selbits0, out0))
    out_ref[0] = sel


def _knn_pallas(xyz, new_xyz):
    return pl.pallas_call(
        _knn_kernel,
        grid=(B, NPOINT // KNN_TILE_P),
        in_specs=[
            pl.BlockSpec((1, 3, N), lambda b, p: (b, 0, 0)),
            pl.BlockSpec((1, 3, KNN_TILE_P), lambda b, p: (b, 0, p)),
        ],
        out_specs=pl.BlockSpec((1, KNN_TILE_P, NSAMPLE), lambda b, p: (b, p, 0)),
        out_shape=jax.ShapeDtypeStruct((B, NPOINT, NSAMPLE), jnp.int32),
    )(xyz, new_xyz)


def _ln_kernel(y_ref, g_ref, b_ref, o_ref):
    y = y_ref[...]
    mu = jnp.mean(y, axis=-1, keepdims=True)
    var = jnp.mean((y - mu) ** 2, axis=-1, keepdims=True)
    o_ref[...] = (y - mu) / jnp.sqrt(var + 1e-5) * g_ref[...] + b_ref[...]


def _layernorm_pallas(y, g, b):
    Bq, P, C = y.shape
    grid = (Bq, P // 256)
    return pl.pallas_call(
        _ln_kernel,
        grid=grid,
        in_specs=[
            pl.BlockSpec((1, 256, C), lambda i, j: (i, j, 0)),
            pl.BlockSpec((C,), lambda i, j: (0,)),
            pl.BlockSpec((C,), lambda i, j: (0,)),
        ],
        out_specs=pl.BlockSpec((1, 256, C), lambda i, j: (i, j, 0)),
        out_shape=jax.ShapeDtypeStruct((Bq, P, C), jnp.float32),
    )(y, g, b)


def kernel(xyz, points, time_stamps, te_w1, te_b1, te_w2, te_b2, tf_w1, tf_b1, tf_w2, tf_b2, c1_w, c1_b, c2_w, c2_b, c3_w, c3_b, bn1_g, bn1_b, bn2_g, bn2_b, bn3_g, bn3_b, q_w, q_b, k_w, k_b, v_w, v_b, o_w, o_b, sw, ln_g, ln_b):
    Bq = xyz.shape[0]
    t = time_stamps[:, 0, :]
    tw = jax.nn.softmax(t * 10.0, axis=-1) + 0.1
    u = jax.random.uniform(jax.random.key(42), t.shape, jnp.float32, 1e-10, 1.0)
    gumbel = -jnp.log(-jnp.log(u))
    idx = jax.lax.top_k(jnp.log(tw) + gumbel, NPOINT)[1]
    new_xyz = jnp.take_along_axis(xyz, idx[:, None, :], axis=2)
    nidx = _knn_pallas(xyz, new_xyz)
    grouped_xyz = jnp.take_along_axis(xyz[:, :, None, :], nidx[:, None, :, :], axis=3)
    grouped_xyz = (grouped_xyz - new_xyz[:, :, :, None]).transpose(0, 1, 3, 2)
    tf = jax.nn.relu(_lin(t[..., None], te_w1, te_b1))
    tf = _lin(tf, te_w2, te_b2)
    div = jnp.exp(jnp.arange(0, TFD, 2, dtype=jnp.float32) * (-(math.log(10000.0) / TFD)))
    pos = t[..., None] * div
    pe = jnp.stack([jnp.sin(pos), jnp.cos(pos)], axis=-1).reshape(Bq, t.shape[1], TFD)
    tf = tf + pe
    comb = jnp.concatenate([points.transpose(0, 2, 1), tf], axis=-1)
    enh = _lin(jax.nn.relu(_lin(comb, tf_w1, tf_b1)), tf_w2, tf_b2).transpose(0, 2, 1)
    gp = jnp.take_along_axis(enh[:, :, None, :], nidx[:, None, :, :], axis=3)
    np_in = jnp.concatenate([grouped_xyz, gp.transpose(0, 1, 3, 2)], axis=1)
    h = jax.nn.relu(_bn(_conv1x1(np_in, c1_w, c1_b), bn1_g, bn1_b))
    h = jax.nn.relu(_bn(_conv1x1(h, c2_w, c2_b), bn2_g, bn2_b))
    h = _bn(_conv1x1(h, c3_w, c3_b), bn3_g, bn3_b)
    pooled = jnp.max(h, axis=2)
    x_in = pooled.transpose(0, 2, 1)
    hd = MLP_CH[2] // HEADS
    outs = []
    for i, scale in enumerate((4, 8, 16)):
        nw = NPOINT // scale
        wx = x_in[:, :nw * scale, :].reshape(Bq, nw, scale, MLP_CH[2]).mean(axis=2)
        q = _lin(wx, q_w, q_b).reshape(Bq, nw, HEADS, hd).transpose(0, 2, 1, 3)
        kk = _lin(wx, k_w, k_b).reshape(Bq, nw, HEADS, hd).transpose(0, 2, 1, 3)
        vv = _lin(wx, v_w, v_b).reshape(Bq, nw, HEADS, hd).transpose(0, 2, 1, 3)
        sc = jnp.einsum('bhid,bhjd->bhij', q, kk) / math.sqrt(hd)
        aw = jax.nn.softmax(sc, axis=-1)
        att = jnp.einsum('bhij,bhjd->bhid', aw, vv).transpose(0, 2, 1, 3).reshape(Bq, nw, MLP_CH[2])
        att = _lin(att, o_w, o_b)
        if nw < NPOINT:
            att = _interp_linear(att.transpose(0, 2, 1), NPOINT).transpose(0, 2, 1)
        outs.append(att * sw[i])
    ms = (outs[0] + outs[1] + outs[2]) / 3.0
    y = x_in + ms
    y = _layernorm_pallas(y, ln_g, ln_b)
    return new_xyz, y.transpose(0, 2, 1)


# speedup vs baseline: 8.1553x; 2.1266x over previous
import math

import jax
import jax.numpy as jnp
from jax.experimental import pallas as pl

B, N, C_IN = 4, 16384, 128
NPOINT, NSAMPLE = 1024, 32
MLP_CH = (128, 128, 256)
TFD = 8
HEADS = 4
M_TOT = B * NPOINT * NSAMPLE

KNN_TILE_P = 128

ENH_TILE = 512
CONV_TILE = 2048


def _knn_kernel(xyz_ref, nxyz_ref, out_ref):
    TILE_P, S = KNN_TILE_P, NSAMPLE
    NCH = 8
    CH = N // NCH
    xyz = xyz_ref[0]
    a = nxyz_ref[0]
    cj2 = jnp.sum(xyz * xyz, axis=0, keepdims=True)
    dots = jax.lax.dot_general(a, xyz, (((0,), (0,)), ((), ())),
                               preferred_element_type=jnp.float32)
    d = cj2 - 2.0 * dots
    D3 = d.reshape(TILE_P, NCH, CH)
    cmin0 = jnp.min(D3, axis=2)
    chunkiota = jax.lax.broadcasted_iota(jnp.int32, (TILE_P, NCH), 1)
    laneiota = jax.lax.broadcasted_iota(jnp.int32, (TILE_P, CH), 1)
    slotiota = jax.lax.broadcasted_iota(jnp.int32, (TILE_P, S), 1)

    def body(i, carry):
        cmin, selbits, out = carry
        cstar = jnp.argmin(cmin, axis=-1).astype(jnp.int32)
        idx3 = jnp.broadcast_to(cstar[:, None, None], (TILE_P, 1, CH))
        contents = jnp.take_along_axis(D3, idx3, axis=1)[:, 0, :]
        hit = jax.lax.shift_right_logical(selbits, cstar[:, None]) & 1
        contents = jnp.where(hit == 1, jnp.inf, contents)
        lstar = jnp.argmin(contents, axis=-1).astype(jnp.int32)
        j = cstar * CH + lstar
        out = jnp.where(slotiota == i, j[:, None], out)
        selbits = selbits | jnp.where(
            laneiota == lstar[:, None],
            jax.lax.shift_left(jnp.ones_like(selbits), cstar[:, None]),
            jnp.zeros_like(selbits))
        c2 = jnp.where(laneiota == lstar[:, None], jnp.inf, contents)
        newmin = jnp.min(c2, axis=-1)
        cmin = jnp.where(chunkiota == cstar[:, None], newmin[:, None], cmin)
        return cmin, selbits, out

    selbits0 = jnp.zeros((TILE_P, CH), jnp.int32)
    out0 = jnp.zeros((TILE_P, S), jnp.int32)
    _, _, sel = jax.lax.fori_loop(0, S, body, (cmin0, selbits0, out0))
    out_ref[0] = sel


def _knn_pallas(xyz, new_xyz):
    return pl.pallas_call(
        _knn_kernel,
        grid=(B, NPOINT // KNN_TILE_P),
        in_specs=[
            pl.BlockSpec((1, 3, N), lambda b, p: (b, 0, 0)),
            pl.BlockSpec((1, 3, KNN_TILE_P), lambda b, p: (b, 0, p)),
        ],
        out_specs=pl.BlockSpec((1, KNN_TILE_P, NSAMPLE), lambda b, p: (b, p, 0)),
        out_shape=jax.ShapeDtypeStruct((B, NPOINT, NSAMPLE), jnp.int32),
    )(xyz, new_xyz)


def _enh_kernel(p_ref, t_ref, tew1_ref, teb1_ref, tew2_ref, teb2_ref,
                tfw1_ref, tfb1_ref, tfw2_ref, tfb2_ref, out_ref):
    pts = p_ref[0]
    t = t_ref[0]
    tf1 = jnp.maximum(t * tew1_ref[...][None, :, 0] + teb1_ref[...][None, :], 0.0)
    tf2 = jax.lax.dot_general(tf1, tew2_ref[...], (((1,), (1,)), ((), ())),
                              preferred_element_type=jnp.float32) + teb2_ref[...][None, :]
    kidx = jax.lax.broadcasted_iota(jnp.int32, (ENH_TILE, TFD), 1)
    freq = jnp.exp((kidx // 2 * 2).astype(jnp.float32) * (-(math.log(10000.0) / TFD)))
    pos = t * freq
    pe = jnp.where(kidx % 2 == 0, jnp.sin(pos), jnp.cos(pos))
    tf = tf2 + pe
    comb = jnp.concatenate([pts, tf], axis=1)
    h = jnp.maximum(jax.lax.dot_general(comb, tfw1_ref[...], (((1,), (1,)), ((), ())),
                                        preferred_element_type=jnp.float32)
                    + tfb1_ref[...][None, :], 0.0)
    out_ref[0] = jax.lax.dot_general(h, tfw2_ref[...], (((1,), (1,)), ((), ())),
                                     preferred_element_type=jnp.float32) + tfb2_ref[...][None, :]


def _enh_pallas(pointsT, tT, te_w1, te_b1, te_w2, te_b2, tf_w1, tf_b1, tf_w2, tf_b2):
    nt = N // ENH_TILE
    return pl.pallas_call(
        _enh_kernel,
        grid=(B, nt),
        in_specs=[
            pl.BlockSpec((1, ENH_TILE, C_IN), lambda b, i: (b, i, 0)),
            pl.BlockSpec((1, ENH_TILE, 1), lambda b, i: (b, i, 0)),
            pl.BlockSpec(te_w1.shape, lambda b, i: (0, 0)),
            pl.BlockSpec(te_b1.shape, lambda b, i: (0,)),
            pl.BlockSpec(te_w2.shape, lambda b, i: (0, 0)),
            pl.BlockSpec(te_b2.shape, lambda b, i: (0,)),
            pl.BlockSpec(tf_w1.shape, lambda b, i: (0, 0)),
            pl.BlockSpec(tf_b1.shape, lambda b, i: (0,)),
            pl.BlockSpec(tf_w2.shape, lambda b, i: (0, 0)),
            pl.BlockSpec(tf_b2.shape, lambda b, i: (0,)),
        ],
        out_specs=pl.BlockSpec((1, ENH_TILE, C_IN), lambda b, i: (b, i, 0)),
        out_shape=jax.ShapeDtypeStruct((B, N, C_IN), jnp.float32),
    )(pointsT, tT, te_w1, te_b1, te_w2, te_b2, tf_w1, tf_b1, tf_w2, tf_b2)


def _acc_stats(z, s_ref, q_ref):
    zg = z.reshape(CONV_TILE // 8, 8, z.shape[-1])

    @pl.when(pl.program_id(0) == 0)
    def _():
        s_ref[...] = jnp.zeros_like(s_ref)
        q_ref[...] = jnp.zeros_like(q_ref)

    s_ref[...] += jnp.sum(zg, axis=0)
    q_ref[...] += jnp.sum(zg * zg, axis=0)


def _bn_affine(s_ref, q_ref, g_ref, b_ref):
    mu = jnp.sum(s_ref[...], axis=0, keepdims=True) / M_TOT
    var = jnp.sum(q_ref[...], axis=0, keepdims=True) / M_TOT - mu * mu
    sc = g_ref[...][None, :] / jnp.sqrt(var + 1e-5)
    sh = b_ref[...][None, :] - mu * sc
    return sc, sh


def _pass1_kernel(gx_ref, gf_ref, nx_ref, w_ref, b_ref, z_ref, s_ref, q_ref):
    ng = CONV_TILE // NSAMPLE
    gx = gx_ref[...]
    nx = nx_ref[...]
    rel = gx - jnp.broadcast_to(nx[:, None, :], (ng, NSAMPLE, 16)).reshape(CONV_TILE, 16)
    x = jnp.concatenate([rel[:, :3], gf_ref[...]], axis=1)
    z = jax.lax.dot_general(x, w_ref[...], (((1,), (1,)), ((), ())),
                            preferred_element_type=jnp.float32) + b_ref[...][None, :]
    z_ref[...] = z
    _acc_stats(z, s_ref, q_ref)


def _passmid_kernel(z_ref, s_ref, q_ref, g_ref, be_ref, w_ref, b_ref,
                    zo_ref, so_ref, qo_ref):
    sc, sh = _bn_affine(s_ref, q_ref, g_ref, be_ref)
    h = jnp.maximum(z_ref[...] * sc + sh, 0.0)
    z = jax.lax.dot_general(h, w_ref[...], (((1,), (1,)), ((), ())),
                            preferred_element_type=jnp.float32) + b_ref[...][None, :]
    zo_ref[...] = z
    _acc_stats(z, so_ref, qo_ref)


def _pass3_kernel(z_ref, s_ref, q_ref, g_ref, be_ref, w_ref, b_ref,
                  zmax_ref, zmin_ref, so_ref, qo_ref):
    sc, sh = _bn_affine(s_ref, q_ref, g_ref, be_ref)
    h = jnp.maximum(z_ref[...] * sc + sh, 0.0)
    z = jax.lax.dot_general(h, w_ref[...], (((1,), (1,)), ((), ())),
                            preferred_element_type=jnp.float32) + b_ref[...][None, :]
    zg = z.reshape(CONV_TILE // NSAMPLE, NSAMPLE, MLP_CH[2])
    zmax_ref[...] = jnp.max(zg, axis=1)
    zmin_ref[...] = jnp.min(zg, axis=1)
    _acc_stats(z, so_ref, qo_ref)


def _conv_stack(gx, gf, nxf, c1_w, c1_b, c2_w, c2_b, c3_w, c3_b,
                bn1_g, bn1_b, bn2_g, bn2_b):
    nsteps = M_TOT // CONV_TILE
    ng = CONV_TILE // NSAMPLE
    z1, s1, q1 = pl.pallas_call(
        _pass1_kernel,
        grid=(nsteps,),
        in_specs=[
            pl.BlockSpec((CONV_TILE, 16), lambda i: (i, 0)),
            pl.BlockSpec((CONV_TILE, C_IN), lambda i: (i, 0)),
            pl.BlockSpec((ng, 16), lambda i: (i, 0)),
            pl.BlockSpec(c1_w.shape, lambda i: (0, 0)),
            pl.BlockSpec(c1_b.shape, lambda i: (0,)),
        ],
        out_specs=[
            pl.BlockSpec((CONV_TILE, MLP_CH[0]), lambda i: (i, 0)),
            pl.BlockSpec((8, MLP_CH[0]), lambda i: (0, 0)),
            pl.BlockSpec((8, MLP_CH[0]), lambda i: (0, 0)),
        ],
        out_shape=[
            jax.ShapeDtypeStruct((M_TOT, MLP_CH[0]), jnp.float32),
            jax.ShapeDtypeStruct((8, MLP_CH[0]), jnp.float32),
            jax.ShapeDtypeStruct((8, MLP_CH[0]), jnp.float32),
        ],
    )(gx, gf, nxf, c1_w, c1_b)

    z2, s2, q2 = pl.pallas_call(
        _passmid_kernel,
        grid=(nsteps,),
        in_specs=[
            pl.BlockSpec((CONV_TILE, MLP_CH[0]), lambda i: (i, 0)),
            pl.BlockSpec((8, MLP_CH[0]), lambda i: (0, 0)),
            pl.BlockSpec((8, MLP_CH[0]), lambda i: (0, 0)),
            pl.BlockSpec(bn1_g.shape, lambda i: (0,)),
            pl.BlockSpec(bn1_b.shape, lambda i: (0,)),
            pl.BlockSpec(c2_w.shape, lambda i: (0, 0)),
            pl.BlockSpec(c2_b.shape, lambda i: (0,)),
        ],
        out_specs=[
            pl.BlockSpec((CONV_TILE, MLP_CH[1]), lambda i: (i, 0)),
            pl.BlockSpec((8, MLP_CH[1]), lambda i: (0, 0)),
            pl.BlockSpec((8, MLP_CH[1]), lambda i: (0, 0)),
        ],
        out_shape=[
            jax.ShapeDtypeStruct((M_TOT, MLP_CH[1]), jnp.float32),
            jax.ShapeDtypeStruct((8, MLP_CH[1]), jnp.float32),
            jax.ShapeDtypeStruct((8, MLP_CH[1]), jnp.float32),
        ],
    )(z1, s1, q1, bn1_g, bn1_b, c2_w, c2_b)

    zmax, zmin, s3, q3 = pl.pallas_call(
        _pass3_kernel,
        grid=(nsteps,),
        in_specs=[
            pl.BlockSpec((CONV_TILE, MLP_CH[1]), lambda i: (i, 0)),
            pl.BlockSpec((8, MLP_CH[1]), lambda i: (0, 0)),
            pl.BlockSpec((8, MLP_CH[1]), lambda i: (0, 0)),
            pl.BlockSpec(bn2_g.shape, lambda i: (0,)),
            pl.BlockSpec(bn2_b.shape, lambda i: (0,)),
            pl.BlockSpec(c3_w.shape, lambda i: (0, 0)),
            pl.BlockSpec(c3_b.shape, lambda i: (0,)),
        ],
        out_specs=[
            pl.BlockSpec((ng, MLP_CH[2]), lambda i: (i, 0)),
            pl.BlockSpec((ng, MLP_CH[2]), lambda i: (i, 0)),
            pl.BlockSpec((8, MLP_CH[2]), lambda i: (0, 0)),
            pl.BlockSpec((8, MLP_CH[2]), lambda i: (0, 0)),
        ],
        out_shape=[
            jax.ShapeDtypeStruct((B * NPOINT, MLP_CH[2]), jnp.float32),
            jax.ShapeDtypeStruct((B * NPOINT, MLP_CH[2]), jnp.float32),
            jax.ShapeDtypeStruct((8, MLP_CH[2]), jnp.float32),
            jax.ShapeDtypeStruct((8, MLP_CH[2]), jnp.float32),
        ],
    )(z2, s2, q2, bn2_g, bn2_b, c3_w, c3_b)
    return zmax, zmin, s3, q3


def _attn_kernel(zmax_ref, zmin_ref, s3_ref, q3_ref, g3_ref, b3_ref,
                 qw_ref, qb_ref, kw_ref, kb_ref, vw_ref, vb_ref,
                 ow3_ref, ob3_ref, lng_ref, lnb_ref, y_ref):
    C = MLP_CH[2]
    hd = C // HEADS
    sc3, sh3 = _bn_affine(s3_ref, q3_ref, g3_ref, b3_ref)
    pooled = jnp.where(sc3 >= 0.0,
                       zmax_ref[0] * sc3 + sh3,
                       zmin_ref[0] * sc3 + sh3)
    acc = jnp.zeros((NPOINT, C), jnp.float32)
    for si, scale in enumerate((4, 8, 16)):
        nw = NPOINT // scale
        rid = jax.lax.broadcasted_iota(jnp.int32, (nw, NPOINT), 0)
        cid = jax.lax.broadcasted_iota(jnp.int32, (nw, NPOINT), 1)
        pm = jnp.where(cid // scale == rid, 1.0 / scale, 0.0)
        wx = jax.lax.dot_general(pm, pooled, (((1,), (0,)), ((), ())),
                                 preferred_element_type=jnp.float32)
        q = jax.lax.dot_general(wx, qw_ref[...], (((1,), (1,)), ((), ())),
                                preferred_element_type=jnp.float32) + qb_ref[...][None, :]
        k = jax.lax.dot_general(wx, kw_ref[...], (((1,), (1,)), ((), ())),
                                preferred_element_type=jnp.float32) + kb_ref[...][None, :]
        v = jax.lax.dot_general(wx, vw_ref[...], (((1,), (1,)), ((), ())),
                                preferred_element_type=jnp.float32) + vb_ref[...][None, :]
        heads = []
        for h in range(HEADS):
            qh = q[:, h * hd:(h + 1) * hd]
            kh = k[:, h * hd:(h + 1) * hd]
            vh = v[:, h * hd:(h + 1) * hd]
            s = jax.lax.dot_general(qh, kh, (((1,), (1,)), ((), ())),
                                    preferred_element_type=jnp.float32) / math.sqrt(hd)
            m = jnp.max(s, axis=-1, keepdims=True)
            e = jnp.exp(s - m)
            aw = e / jnp.sum(e, axis=-1, keepdims=True)
            heads.append(jax.lax.dot_general(aw, vh, (((1,), (0,)), ((), ())),
                                             preferred_element_type=jnp.float32))
        att = jnp.concatenate(heads, axis=1)
        att = jax.lax.dot_general(att, ow3_ref[si], (((1,), (1,)), ((), ())),
                                  preferred_element_type=jnp.float32) + ob3_ref[...][si][None, :]
        r = jax.lax.broadcasted_iota(jnp.int32, (NPOINT, nw), 0).astype(jnp.float32)
        c = jax.lax.broadcasted_iota(jnp.int32, (NPOINT, nw), 1).astype(jnp.float32)
        src = jnp.clip((r + 0.5) * (nw / NPOINT) - 0.5, 0.0, nw - 1.0)
        i0 = jnp.floor(src)
        i1 = jnp.minimum(i0 + 1.0, nw - 1.0)
        w = src - i0
        wi = jnp.where(c == i0, 1.0 - w, 0.0) + jnp.where(c == i1, w, 0.0)
        acc += jax.lax.dot_general(wi, att, (((1,), (0,)), ((), ())),
                                   preferred_element_type=jnp.float32)
    y = pooled + acc
    mu = jnp.mean(y, axis=-1, keepdims=True)
    var = jnp.mean((y - mu) ** 2, axis=-1, keepdims=True)
    y_ref[0] = (y - mu) / jnp.sqrt(var + 1e-5) * lng_ref[...][None, :] + lnb_ref[...][None, :]


def _attn_pallas(zmax, zmin, s3, q3, bn3_g, bn3_b, q_w, q_b, k_w, k_b,
                 v_w, v_b, o_w3, o_b3, ln_g, ln_b):
    C = MLP_CH[2]
    return pl.pallas_call(
        _attn_kernel,
        grid=(B,),
        in_specs=[
            pl.BlockSpec((1, NPOINT, C), lambda b: (b, 0, 0)),
            pl.BlockSpec((1, NPOINT, C), lambda b: (b, 0, 0)),
            pl.BlockSpec((8, C), lambda b: (0, 0)),
            pl.BlockSpec((8, C), lambda b: (0, 0)),
            pl.BlockSpec((C,), lambda b: (0,)),
            pl.BlockSpec((C,), lambda b: (0,)),
            pl.BlockSpec((C, C), lambda b: (0, 0)),
            pl.BlockSpec((C,), lambda b: (0,)),
            pl.BlockSpec((C, C), lambda b: (0, 0)),
            pl.BlockSpec((C,), lambda b: (0,)),
            pl.BlockSpec((C, C), lambda b: (0, 0)),
            pl.BlockSpec((C,), lambda b: (0,)),
            pl.BlockSpec((3, C, C), lambda b: (0, 0, 0)),
            pl.BlockSpec((3, C), lambda b: (0, 0)),
            pl.BlockSpec((C,), lambda b: (0,)),
            pl.BlockSpec((C,), lambda b: (0,)),
        ],
        out_specs=pl.BlockSpec((1, NPOINT, C), lambda b: (b, 0, 0)),
        out_shape=jax.ShapeDtypeStruct((B, NPOINT, C), jnp.float32),
    )(zmax, zmin, s3, q3, bn3_g, bn3_b, q_w, q_b, k_w, k_b, v_w, v_b,
      o_w3, o_b3, ln_g, ln_b)


def kernel(xyz, points, time_stamps, te_w1, te_b1, te_w2, te_b2, tf_w1, tf_b1, tf_w2, tf_b2, c1_w, c1_b, c2_w, c2_b, c3_w, c3_b, bn1_g, bn1_b, bn2_g, bn2_b, bn3_g, bn3_b, q_w, q_b, k_w, k_b, v_w, v_b, o_w, o_b, sw, ln_g, ln_b):
    t = time_stamps[:, 0, :]
    tw = jax.nn.softmax(t * 10.0, axis=-1) + 0.1
    u = jax.random.uniform(jax.random.key(42), t.shape, jnp.float32, 1e-10, 1.0)
    gumbel = -jnp.log(-jnp.log(u))
    idx = jax.lax.top_k(jnp.log(tw) + gumbel, NPOINT)[1]
    new_xyz = jnp.take_along_axis(xyz, idx[:, None, :], axis=2)

    nidx = _knn_pallas(xyz, new_xyz)

    pointsT = points.transpose(0, 2, 1)
    tT = time_stamps.transpose(0, 2, 1)
    enh = _enh_pallas(pointsT, tT, te_w1, te_b1, te_w2, te_b2,
                      tf_w1, tf_b1, tf_w2, tf_b2)

    fidx = (nidx + (jnp.arange(B, dtype=jnp.int32) * N)[:, None, None]).reshape(-1)
    gf = jnp.take(enh.reshape(B * N, C_IN), fidx, axis=0)
    xyzT = xyz.transpose(0, 2, 1)
    xyz16 = jnp.pad(xyzT, ((0, 0), (0, 0), (0, 13))).reshape(B * N, 16)
    gx = jnp.take(xyz16, fidx, axis=0)
    nxf = jnp.pad(new_xyz.transpose(0, 2, 1), ((0, 0), (0, 0), (0, 13))).reshape(B * NPOINT, 16)

    zmax, zmin, s3, q3 = _conv_stack(gx, gf, nxf, c1_w, c1_b, c2_w, c2_b,
                                     c3_w, c3_b, bn1_g, bn1_b, bn2_g, bn2_b)

    o_w3 = o_w[None, :, :] * (sw / 3.0)[:, None, None]
    o_b3 = o_b[None, :] * (sw / 3.0)[:, None]
    y = _attn_pallas(zmax.reshape(B, NPOINT, MLP_CH[2]),
                     zmin.reshape(B, NPOINT, MLP_CH[2]),
                     s3, q3, bn3_g, bn3_b, q_w, q_b, k_w, k_b, v_w, v_b,
                     o_w3, o_b3, ln_g, ln_b)
    return new_xyz, y.transpose(0, 2, 1)
